# baseline (speedup 1.0000x reference)
import functools

import jax
import jax.numpy as jnp
from jax import lax
from jax.experimental import pallas as pl
from jax.experimental.pallas import tpu as pltpu
from jax.experimental.pallas import tpu_sc as plsc

NC = 2
NS = 16
NW = NC * NS
L = 16
BATCH = 16384
DIM = 16
BPW = BATCH // NW
NCHUNK = 4
CHUNK = BPW // NCHUNK


def _body(u_hbm, i_hbm, ut_hbm, it_hbm, out_hbm,
          idx_u, idx_i, ue, ie, out_v, sem):
    wid = lax.axis_index("s") * NC + lax.axis_index("c")
    base = wid * BPW

    pltpu.sync_copy(u_hbm.at[wid], idx_u)
    pltpu.sync_copy(i_hbm.at[wid], idx_i)

    copies = []
    for j in range(NCHUNK):
        copies.append(pltpu.async_copy(
            ut_hbm.at[idx_u.at[j]], ue.at[pl.ds(j * CHUNK, CHUNK)], sem))
        copies.append(pltpu.async_copy(
            it_hbm.at[idx_i.at[j]], ie.at[pl.ds(j * CHUNK, CHUNK)], sem))
    for c in copies:
        c.wait()

    def group(g, carry):
        r0 = g * L
        rows = lax.iota(jnp.int32, L) + r0
        acc = jnp.zeros((L,), jnp.float32)
        for d in range(DIM):
            col = jnp.full((L,), d, jnp.int32)
            uc = plsc.load_gather(ue, [rows, col])
            ic = plsc.load_gather(ie, [rows, col])
            acc = acc + uc * ic
        out_v[pl.ds(r0, L)] = acc
        return carry

    lax.fori_loop(0, BPW // L, group, 0)

    pltpu.sync_copy(out_v, out_hbm.at[pl.ds(base, BPW)])


@jax.jit
def kernel(u, i, user_table, item_table):
    u3 = u.astype(jnp.int32).reshape(NW, NCHUNK, CHUNK)
    i3 = i.astype(jnp.int32).reshape(NW, NCHUNK, CHUNK)
    mesh = plsc.VectorSubcoreMesh(core_axis_name="c", subcore_axis_name="s")
    f = pl.kernel(
        _body,
        out_type=jax.ShapeDtypeStruct((BATCH,), jnp.float32),
        mesh=mesh,
        compiler_params=pltpu.CompilerParams(
            needs_layout_passes=False, use_tc_tiling_on_sc=False),
        scratch_types=[
            pltpu.VMEM((NCHUNK, CHUNK), jnp.int32),
            pltpu.VMEM((NCHUNK, CHUNK), jnp.int32),
            pltpu.VMEM((BPW, DIM), jnp.float32),
            pltpu.VMEM((BPW, DIM), jnp.float32),
            pltpu.VMEM((BPW,), jnp.float32),
            pltpu.SemaphoreType.DMA,
        ],
    )
    return f(u3, i3, user_table, item_table)
